# baseline (speedup 1.0000x reference)
import functools

import jax
import jax.numpy as jnp
from jax import lax
from jax.experimental import pallas as pl
from jax.experimental.pallas import tpu as pltpu
from jax.experimental.pallas import tpu_sc as plsc

EMB_DIM = 64
BATCH = 4096
HIST = 200
SPLIT = 128
REM = HIST - SPLIT

_INFO = plsc.get_sparse_core_info()
NC = _INFO.num_cores
NS = _INFO.num_subcores
NW = NC * NS
ROWS_W = BATCH // NW
NB = 4
NG = ROWS_W // NB


def _gather_body(idx_hbm, table_hbm, out_hbm, idx_v, buf,
                 sem_g0, sem_g1, sem_g2, sem_g3,
                 sem_s0, sem_s1, sem_s2, sem_s3):
    sem_g = (sem_g0, sem_g1, sem_g2, sem_g3)
    sem_s = (sem_s0, sem_s1, sem_s2, sem_s3)
    c = lax.axis_index("c")
    s = lax.axis_index("s")
    wid = s * NC + c
    base_row = wid * ROWS_W

    pltpu.sync_copy(idx_hbm.at[pl.ds(base_row, ROWS_W)], idx_v)

    def fire_gathers(b, r):
        pltpu.async_copy(
            table_hbm.at[idx_v.at[r, pl.ds(0, SPLIT)]],
            buf.at[b, pl.ds(0, SPLIT)],
            sem_g[b],
        )
        pltpu.async_copy(
            table_hbm.at[idx_v.at[r, pl.ds(SPLIT, REM)]],
            buf.at[b, pl.ds(SPLIT, REM)],
            sem_g[b],
        )

    def wait_gathers(b):
        pltpu.make_async_copy(
            table_hbm.at[idx_v.at[0, pl.ds(0, SPLIT)]],
            buf.at[b, pl.ds(0, SPLIT)], sem_g[b],
        ).wait()
        pltpu.make_async_copy(
            table_hbm.at[idx_v.at[0, pl.ds(SPLIT, REM)]],
            buf.at[b, pl.ds(SPLIT, REM)], sem_g[b],
        ).wait()

    for b in range(NB):
        fire_gathers(b, b)

    def round_body(g, carry):
        for b in range(NB):
            r = g * NB + b
            wait_gathers(b)
            pltpu.async_copy(buf.at[b], out_hbm.at[base_row + r], sem_s[b])
            pltpu.make_async_copy(
                buf.at[b], out_hbm.at[base_row], sem_s[b],
            ).wait()

            @pl.when(r + NB < ROWS_W)
            def _():
                fire_gathers(b, r + NB)
        return carry

    lax.fori_loop(0, NG, round_body, 0)


@functools.partial(
    pl.kernel,
    out_type=jax.ShapeDtypeStruct((BATCH, HIST, EMB_DIM), jnp.float32),
    mesh=plsc.VectorSubcoreMesh(core_axis_name="c", subcore_axis_name="s"),
    scratch_types=[
        pltpu.VMEM((ROWS_W, HIST), jnp.int32),
        pltpu.VMEM((NB, HIST, EMB_DIM), jnp.float32),
        pltpu.SemaphoreType.DMA,
        pltpu.SemaphoreType.DMA,
        pltpu.SemaphoreType.DMA,
        pltpu.SemaphoreType.DMA,
        pltpu.SemaphoreType.DMA,
        pltpu.SemaphoreType.DMA,
        pltpu.SemaphoreType.DMA,
        pltpu.SemaphoreType.DMA,
    ],
    compiler_params=pltpu.CompilerParams(use_tc_tiling_on_sc=False),
)
def _gather_kernel(idx_hbm, table_hbm, out_hbm, idx_v, buf,
                   sem_g0, sem_g1, sem_g2, sem_g3,
                   sem_s0, sem_s1, sem_s2, sem_s3):
    _gather_body(idx_hbm, table_hbm, out_hbm, idx_v, buf,
                 sem_g0, sem_g1, sem_g2, sem_g3,
                 sem_s0, sem_s1, sem_s2, sem_s3)


def kernel(indices, table):
    return _gather_kernel(indices, table)
